# baseline (speedup 1.0000x reference)
import functools

import jax
import jax.numpy as jnp
from jax import lax
from jax.experimental import pallas as pl
from jax.experimental.pallas import tpu as pltpu
from jax.experimental.pallas import tpu_sc as plsc

N_NODES = 10000
N_EDGES = 320000
D = 128

NC = 2
NS = 16
NW = NC * NS
EPW = N_EDGES // NW
CH = 80
NCH = EPW // CH
NB = 2
CSC = 25
SCH = NCH // CSC
NITER = -(-CSC // NB)
RPT = 624
TAIL = N_NODES - RPT * NS
ZR = 48

_mesh = plsc.VectorSubcoreMesh(core_axis_name="c", subcore_axis_name="s")


@functools.partial(
    pl.kernel,
    out_type=jax.ShapeDtypeStruct((NC, N_NODES, D), jnp.float32),
    mesh=_mesh,
    scratch_types=[
        pltpu.VMEM((CSC, CH), jnp.int32),
        pltpu.VMEM((CSC, CH), jnp.int32),
        pltpu.VMEM((NB, CH, D), jnp.float32),
        pltpu.VMEM((ZR, D), jnp.float32),
        pltpu.VMEM_SHARED((N_NODES, D), jnp.float32),
        pltpu.SemaphoreType.DMA((NB,)),
        pltpu.SemaphoreType.DMA((NB,)),
    ],
)
def _sc_agg(table, src, dst, out, sbuf, dbuf, rows, zbuf, acc, gsem, ssem):
    cid = lax.axis_index("c")
    sid = lax.axis_index("s")
    wid = sid * NC + cid

    zv = jnp.zeros((16,), jnp.float32)
    for i in range(ZR):
        for j in range(D // 16):
            zbuf[i, pl.ds(j * 16, 16)] = zv

    def zero_body(i, carry):
        pltpu.sync_copy(zbuf, acc.at[pl.ds(sid * RPT + i * ZR, ZR)])
        return carry

    lax.fori_loop(0, RPT // ZR, zero_body, 0)

    @pl.when(sid == 0)
    def _zero_tail():
        pltpu.sync_copy(zbuf.at[pl.ds(0, TAIL)],
                        acc.at[pl.ds(RPT * NS, TAIL)])

    plsc.subcore_barrier()

    def super_body(s, carry):
        pltpu.sync_copy(src.at[wid, s], sbuf)
        pltpu.sync_copy(dst.at[wid, s], dbuf)

        for b in range(NB):
            pltpu.async_copy(table.at[sbuf.at[b]], rows.at[b], gsem.at[b])

        def main(i, carry2):
            for b in range(NB):
                j = i * NB + b

                @pl.when(j < CSC)
                def _scatter(b=b, j=j):
                    pltpu.make_async_copy(table.at[sbuf.at[j]], rows.at[b],
                                          gsem.at[b]).wait()
                    pltpu.async_copy(rows.at[b], acc.at[dbuf.at[j]],
                                     ssem.at[b], add=True)

            for b in range(NB):
                j = i * NB + b

                @pl.when(j < CSC)
                def _drain(b=b, j=j):
                    pltpu.make_async_copy(rows.at[b], acc.at[dbuf.at[j]],
                                          ssem.at[b]).wait()

                @pl.when(j + NB < CSC)
                def _prefetch(b=b, j=j):
                    pltpu.async_copy(table.at[sbuf.at[j + NB]], rows.at[b],
                                     gsem.at[b])

            return carry2

        lax.fori_loop(0, NITER, main, 0)
        return carry

    lax.fori_loop(0, SCH, super_body, 0)
    plsc.subcore_barrier()

    pltpu.sync_copy(acc.at[pl.ds(sid * RPT, RPT)],
                    out.at[cid, pl.ds(sid * RPT, RPT)])

    @pl.when(sid == 0)
    def _copy_tail():
        pltpu.sync_copy(acc.at[pl.ds(RPT * NS, TAIL)],
                        out.at[cid, pl.ds(RPT * NS, TAIL)])


def _dense_body(p_ref, x_ref, wrel_ref, wroot_ref, b_ref, o_ref, *, relu):
    agg = p_ref[0] + p_ref[1]
    acc = jnp.dot(agg, wrel_ref[...], preferred_element_type=jnp.float32)
    acc = acc + jnp.dot(x_ref[...], wroot_ref[...],
                        preferred_element_type=jnp.float32)
    acc = acc + b_ref[...]
    o_ref[...] = jnp.maximum(acc, 0.0) if relu else acc


def _dense(partials, x, w_rel, w_root, b, relu):
    bn = 2000
    grid = (N_NODES // bn,)
    return pl.pallas_call(
        functools.partial(_dense_body, relu=relu),
        grid=grid,
        in_specs=[
            pl.BlockSpec((NC, bn, D), lambda i: (0, i, 0)),
            pl.BlockSpec((bn, D), lambda i: (i, 0)),
            pl.BlockSpec((D, D), lambda i: (0, 0)),
            pl.BlockSpec((D, D), lambda i: (0, 0)),
            pl.BlockSpec((1, D), lambda i: (0, 0)),
        ],
        out_specs=pl.BlockSpec((bn, D), lambda i: (i, 0)),
        out_shape=jax.ShapeDtypeStruct((N_NODES, D), jnp.float32),
    )(partials, x, w_rel, w_root, b.reshape(1, D))


def kernel(x, edge_index, W1_rel, b1, W1_root, W2_rel, b2, W2_root):
    ei = edge_index.astype(jnp.int32)
    src = ei[0].reshape(NW, SCH, CSC, CH)
    dst = ei[1].reshape(NW, SCH, CSC, CH)
    p1 = _sc_agg(x, src, dst)
    h = _dense(p1, x, W1_rel, W1_root, b1, relu=True)
    p2 = _sc_agg(h, src, dst)
    out = _dense(p2, h, W2_rel, W2_root, b2, relu=False)
    return out


# speedup vs baseline: 1.2554x; 1.2554x over previous
import functools

import jax
import jax.numpy as jnp
from jax import lax
from jax.experimental import pallas as pl
from jax.experimental.pallas import tpu as pltpu
from jax.experimental.pallas import tpu_sc as plsc

N_NODES = 10000
N_EDGES = 320000
D = 128

NC = 2
NS = 16
NW = NC * NS
EPW = N_EDGES // NW
CH = 40
NCH = EPW // CH
NB = 6
CSC = 50
SCH = NCH // CSC
NITER = -(-CSC // NB)
RPT = 624
TAIL = N_NODES - RPT * NS
ZR = 16

_mesh = plsc.VectorSubcoreMesh(core_axis_name="c", subcore_axis_name="s")


@functools.partial(
    pl.kernel,
    out_type=jax.ShapeDtypeStruct((NC, N_NODES, D), jnp.float32),
    mesh=_mesh,
    scratch_types=[
        pltpu.VMEM((CSC, CH), jnp.int32),
        pltpu.VMEM((CSC, CH), jnp.int32),
        pltpu.VMEM((NB, CH, D), jnp.float32),
        pltpu.VMEM((ZR, D), jnp.float32),
        pltpu.VMEM_SHARED((N_NODES, D), jnp.float32),
        pltpu.SemaphoreType.DMA((NB,)),
        pltpu.SemaphoreType.DMA((NB,)),
    ],
)
def _sc_agg(table, src, dst, out, sbuf, dbuf, rows, zbuf, acc, gsem, ssem):
    cid = lax.axis_index("c")
    sid = lax.axis_index("s")
    wid = sid * NC + cid

    zv = jnp.zeros((16,), jnp.float32)
    for i in range(ZR):
        for j in range(D // 16):
            zbuf[i, pl.ds(j * 16, 16)] = zv

    def zero_body(i, carry):
        pltpu.sync_copy(zbuf, acc.at[pl.ds(sid * RPT + i * ZR, ZR)])
        return carry

    lax.fori_loop(0, RPT // ZR, zero_body, 0)

    @pl.when(sid == 0)
    def _zero_tail():
        pltpu.sync_copy(zbuf.at[pl.ds(0, TAIL)],
                        acc.at[pl.ds(RPT * NS, TAIL)])

    plsc.subcore_barrier()

    def super_body(s, carry):
        pltpu.sync_copy(src.at[wid, s], sbuf)
        pltpu.sync_copy(dst.at[wid, s], dbuf)

        for b in range(NB):
            pltpu.async_copy(table.at[sbuf.at[b]], rows.at[b], gsem.at[b])

        def main(i, carry2):
            for b in range(NB):
                j = i * NB + b

                @pl.when(j < CSC)
                def _scatter(b=b, j=j):
                    pltpu.make_async_copy(table.at[sbuf.at[j]], rows.at[b],
                                          gsem.at[b]).wait()
                    pltpu.async_copy(rows.at[b], acc.at[dbuf.at[j]],
                                     ssem.at[b], add=True)

            for b in range(NB):
                j = i * NB + b

                @pl.when(j < CSC)
                def _drain(b=b, j=j):
                    pltpu.make_async_copy(rows.at[b], acc.at[dbuf.at[j]],
                                          ssem.at[b]).wait()

                @pl.when(j + NB < CSC)
                def _prefetch(b=b, j=j):
                    pltpu.async_copy(table.at[sbuf.at[j + NB]], rows.at[b],
                                     gsem.at[b])

            return carry2

        lax.fori_loop(0, NITER, main, 0)
        return carry

    lax.fori_loop(0, SCH, super_body, 0)
    plsc.subcore_barrier()

    pltpu.sync_copy(acc.at[pl.ds(sid * RPT, RPT)],
                    out.at[cid, pl.ds(sid * RPT, RPT)])

    @pl.when(sid == 0)
    def _copy_tail():
        pltpu.sync_copy(acc.at[pl.ds(RPT * NS, TAIL)],
                        out.at[cid, pl.ds(RPT * NS, TAIL)])


def _dense_body(p_ref, x_ref, wrel_ref, wroot_ref, b_ref, o_ref, *, relu):
    agg = p_ref[0] + p_ref[1]
    acc = jnp.dot(agg, wrel_ref[...], preferred_element_type=jnp.float32)
    acc = acc + jnp.dot(x_ref[...], wroot_ref[...],
                        preferred_element_type=jnp.float32)
    acc = acc + b_ref[...]
    o_ref[...] = jnp.maximum(acc, 0.0) if relu else acc


def _dense(partials, x, w_rel, w_root, b, relu):
    bn = 2000
    grid = (N_NODES // bn,)
    return pl.pallas_call(
        functools.partial(_dense_body, relu=relu),
        grid=grid,
        in_specs=[
            pl.BlockSpec((NC, bn, D), lambda i: (0, i, 0)),
            pl.BlockSpec((bn, D), lambda i: (i, 0)),
            pl.BlockSpec((D, D), lambda i: (0, 0)),
            pl.BlockSpec((D, D), lambda i: (0, 0)),
            pl.BlockSpec((1, D), lambda i: (0, 0)),
        ],
        out_specs=pl.BlockSpec((bn, D), lambda i: (i, 0)),
        out_shape=jax.ShapeDtypeStruct((N_NODES, D), jnp.float32),
    )(partials, x, w_rel, w_root, b.reshape(1, D))


def kernel(x, edge_index, W1_rel, b1, W1_root, W2_rel, b2, W2_root):
    ei = edge_index.astype(jnp.int32)
    src = ei[0].reshape(NW, SCH, CSC, CH)
    dst = ei[1].reshape(NW, SCH, CSC, CH)
    p1 = _sc_agg(x, src, dst)
    h = _dense(p1, x, W1_rel, W1_root, b1, relu=True)
    p2 = _sc_agg(h, src, dst)
    out = _dense(p2, h, W2_rel, W2_root, b2, relu=False)
    return out


# speedup vs baseline: 1.2568x; 1.0012x over previous
import functools

import jax
import jax.numpy as jnp
from jax import lax
from jax.experimental import pallas as pl
from jax.experimental.pallas import tpu as pltpu
from jax.experimental.pallas import tpu_sc as plsc

N_NODES = 10000
N_EDGES = 320000
D = 128

NC = 2
NS = 16
NW = NC * NS
EPW = N_EDGES // NW
CH = 40
NCH = EPW // CH
NB = 6
CSC = 50
SCH = NCH // CSC
NITER = -(-CSC // NB)
RPT = 624
TAIL = N_NODES - RPT * NS
ZR = 16

_mesh = plsc.VectorSubcoreMesh(core_axis_name="c", subcore_axis_name="s")


@functools.partial(
    pl.kernel,
    out_type=jax.ShapeDtypeStruct((NC, N_NODES, D), jnp.float32),
    mesh=_mesh,
    scratch_types=[
        pltpu.VMEM((CSC, CH), jnp.int32),
        pltpu.VMEM((CSC, CH), jnp.int32),
        pltpu.VMEM((NB, CH, D), jnp.float32),
        pltpu.VMEM((ZR, D), jnp.float32),
        pltpu.VMEM_SHARED((N_NODES, D), jnp.float32),
        pltpu.SemaphoreType.DMA((NB,)),
        pltpu.SemaphoreType.DMA((NB,)),
    ],
)
def _sc_agg(table, src, dst, out, sbuf, dbuf, rows, zbuf, acc, gsem, ssem):
    cid = lax.axis_index("c")
    sid = lax.axis_index("s")
    wid = sid * NC + cid

    zv = jnp.zeros((16,), jnp.float32)
    for i in range(ZR):
        for j in range(D // 16):
            zbuf[i, pl.ds(j * 16, 16)] = zv

    def zero_body(i, carry):
        pltpu.sync_copy(zbuf, acc.at[pl.ds(sid * RPT + i * ZR, ZR)])
        return carry

    lax.fori_loop(0, RPT // ZR, zero_body, 0)

    @pl.when(sid == 0)
    def _zero_tail():
        pltpu.sync_copy(zbuf.at[pl.ds(0, TAIL)],
                        acc.at[pl.ds(RPT * NS, TAIL)])

    plsc.subcore_barrier()

    def super_body(s, carry):
        pltpu.sync_copy(src.at[wid, s], sbuf)
        pltpu.sync_copy(dst.at[wid, s], dbuf)

        for b in range(NB):
            pltpu.async_copy(table.at[sbuf.at[b]], rows.at[b], gsem.at[b])

        def main(i, carry2):
            for b in range(NB):
                j = i * NB + b

                @pl.when(j < CSC)
                def _scatter(b=b, j=j):
                    pltpu.make_async_copy(table.at[sbuf.at[j]], rows.at[b],
                                          gsem.at[b]).wait()
                    pltpu.async_copy(rows.at[b], acc.at[dbuf.at[j]],
                                     ssem.at[b], add=True)

            for b in range(NB):
                j = i * NB + b

                @pl.when(j < CSC)
                def _drain(b=b, j=j):
                    pltpu.make_async_copy(rows.at[b], acc.at[dbuf.at[j]],
                                          ssem.at[b]).wait()

                @pl.when(j + NB < CSC)
                def _prefetch(b=b, j=j):
                    pltpu.async_copy(table.at[sbuf.at[j + NB]], rows.at[b],
                                     gsem.at[b])

            return carry2

        lax.fori_loop(0, NITER, main, 0)
        return carry

    lax.fori_loop(0, SCH, super_body, 0)
    plsc.subcore_barrier()

    pltpu.sync_copy(acc.at[pl.ds(sid * RPT, RPT)],
                    out.at[cid, pl.ds(sid * RPT, RPT)])

    @pl.when(sid == 0)
    def _copy_tail():
        pltpu.sync_copy(acc.at[pl.ds(RPT * NS, TAIL)],
                        out.at[cid, pl.ds(RPT * NS, TAIL)])


_BN = 2000


def _root_body(x_ref, w_ref, b_ref, o_ref):
    o_ref[...] = jnp.dot(x_ref[...], w_ref[...],
                         preferred_element_type=jnp.float32) + b_ref[...]


def _root(x, w_root, b):
    return pl.pallas_call(
        _root_body,
        grid=(N_NODES // _BN,),
        in_specs=[
            pl.BlockSpec((_BN, D), lambda i: (i, 0)),
            pl.BlockSpec((D, D), lambda i: (0, 0)),
            pl.BlockSpec((1, D), lambda i: (0, 0)),
        ],
        out_specs=pl.BlockSpec((_BN, D), lambda i: (i, 0)),
        out_shape=jax.ShapeDtypeStruct((N_NODES, D), jnp.float32),
    )(x, w_root, b.reshape(1, D))


def _post_body(p_ref, r_ref, wrel_ref, o_ref, *, relu):
    agg = p_ref[0] + p_ref[1]
    acc = jnp.dot(agg, wrel_ref[...],
                  preferred_element_type=jnp.float32) + r_ref[...]
    o_ref[...] = jnp.maximum(acc, 0.0) if relu else acc


def _post(partials, root, w_rel, relu):
    return pl.pallas_call(
        functools.partial(_post_body, relu=relu),
        grid=(N_NODES // _BN,),
        in_specs=[
            pl.BlockSpec((NC, _BN, D), lambda i: (0, i, 0)),
            pl.BlockSpec((_BN, D), lambda i: (i, 0)),
            pl.BlockSpec((D, D), lambda i: (0, 0)),
        ],
        out_specs=pl.BlockSpec((_BN, D), lambda i: (i, 0)),
        out_shape=jax.ShapeDtypeStruct((N_NODES, D), jnp.float32),
    )(partials, root, w_rel)


def kernel(x, edge_index, W1_rel, b1, W1_root, W2_rel, b2, W2_root):
    ei = edge_index.astype(jnp.int32)
    src = ei[0].reshape(NW, SCH, CSC, CH)
    dst = ei[1].reshape(NW, SCH, CSC, CH)
    p1 = _sc_agg(x, src, dst)
    root1 = _root(x, W1_root, b1)
    h = _post(p1, root1, W1_rel, relu=True)
    p2 = _sc_agg(h, src, dst)
    root2 = _root(h, W2_root, b2)
    out = _post(p2, root2, W2_rel, relu=False)
    return out


# speedup vs baseline: 1.2844x; 1.0219x over previous
import functools

import jax
import jax.numpy as jnp
from jax import lax
from jax.experimental import pallas as pl
from jax.experimental.pallas import tpu as pltpu
from jax.experimental.pallas import tpu_sc as plsc

N_NODES = 10000
N_EDGES = 320000
D = 128

NC = 2
NS = 16
NW = NC * NS
EPW = N_EDGES // NW
CH = 40
NCH = EPW // CH
NB = 6
CSC = 25
SCH = NCH // CSC
NITER = -(-CSC // NB)
RPT = 624
TAIL = N_NODES - RPT * NS
ZR = 16

_mesh = plsc.VectorSubcoreMesh(core_axis_name="c", subcore_axis_name="s")


@functools.partial(
    pl.kernel,
    out_type=jax.ShapeDtypeStruct((NC, N_NODES, D), jnp.float32),
    mesh=_mesh,
    scratch_types=[
        pltpu.VMEM((2, CSC, CH), jnp.int32),
        pltpu.VMEM((2, CSC, CH), jnp.int32),
        pltpu.VMEM((NB, CH, D), jnp.float32),
        pltpu.VMEM((ZR, D), jnp.float32),
        pltpu.VMEM_SHARED((N_NODES, D), jnp.float32),
        pltpu.SemaphoreType.DMA((NB,)),
        pltpu.SemaphoreType.DMA((NB,)),
        pltpu.SemaphoreType.DMA,
        pltpu.SemaphoreType.DMA,
    ],
)
def _sc_agg(table, src, dst, out, sbuf, dbuf, rows, zbuf, acc, gsem, ssem,
            zsem, isem):
    cid = lax.axis_index("c")
    sid = lax.axis_index("s")
    wid = sid * NC + cid

    zv = jnp.zeros((16,), jnp.float32)
    for i in range(ZR):
        for j in range(D // 16):
            zbuf[i, pl.ds(j * 16, 16)] = zv

    def zero_issue(i, carry):
        pltpu.async_copy(zbuf, acc.at[pl.ds(sid * RPT + i * ZR, ZR)], zsem)
        return carry

    lax.fori_loop(0, RPT // ZR, zero_issue, 0)

    @pl.when(sid == 0)
    def _zero_tail():
        pltpu.async_copy(zbuf.at[pl.ds(0, TAIL)],
                         acc.at[pl.ds(RPT * NS, TAIL)], zsem)

    pltpu.sync_copy(src.at[wid, 0], sbuf.at[0])
    pltpu.sync_copy(dst.at[wid, 0], dbuf.at[0])
    for b in range(NB):
        pltpu.async_copy(table.at[sbuf.at[0, b]], rows.at[b], gsem.at[b])

    def zero_drain(i, carry):
        pltpu.make_async_copy(zbuf, acc.at[pl.ds(sid * RPT + i * ZR, ZR)],
                              zsem).wait()
        return carry

    lax.fori_loop(0, RPT // ZR, zero_drain, 0)

    @pl.when(sid == 0)
    def _zero_tail_drain():
        pltpu.make_async_copy(zbuf.at[pl.ds(0, TAIL)],
                              acc.at[pl.ds(RPT * NS, TAIL)], zsem).wait()

    plsc.subcore_barrier()

    def super_body(s, carry):
        p = s % 2

        @pl.when(s + 1 < SCH)
        def _stage_next():
            pltpu.async_copy(src.at[wid, s + 1], sbuf.at[1 - p], isem)
            pltpu.async_copy(dst.at[wid, s + 1], dbuf.at[1 - p], isem)

        def main(i, carry2):
            for b in range(NB):
                j = i * NB + b

                @pl.when(j < CSC)
                def _scatter(b=b, j=j):
                    pltpu.make_async_copy(table.at[sbuf.at[p, j]], rows.at[b],
                                          gsem.at[b]).wait()
                    pltpu.async_copy(rows.at[b], acc.at[dbuf.at[p, j]],
                                     ssem.at[b], add=True)

            for b in range(NB):
                j = i * NB + b

                @pl.when(j < CSC)
                def _drain(b=b, j=j):
                    pltpu.make_async_copy(rows.at[b], acc.at[dbuf.at[p, j]],
                                          ssem.at[b]).wait()

                @pl.when(j + NB < CSC)
                def _prefetch(b=b, j=j):
                    pltpu.async_copy(table.at[sbuf.at[p, j + NB]], rows.at[b],
                                     gsem.at[b])

            return carry2

        lax.fori_loop(0, NITER, main, 0)

        @pl.when(s + 1 < SCH)
        def _prime_next():
            pltpu.make_async_copy(src.at[wid, s + 1], sbuf.at[1 - p],
                                  isem).wait()
            pltpu.make_async_copy(dst.at[wid, s + 1], dbuf.at[1 - p],
                                  isem).wait()
            for b in range(NB):
                pltpu.async_copy(table.at[sbuf.at[1 - p, b]], rows.at[b],
                                 gsem.at[b])
        return carry

    lax.fori_loop(0, SCH, super_body, 0)
    plsc.subcore_barrier()

    pltpu.sync_copy(acc.at[pl.ds(sid * RPT, RPT)],
                    out.at[cid, pl.ds(sid * RPT, RPT)])

    @pl.when(sid == 0)
    def _copy_tail():
        pltpu.sync_copy(acc.at[pl.ds(RPT * NS, TAIL)],
                        out.at[cid, pl.ds(RPT * NS, TAIL)])


_BN = 2000


def _root_body(x_ref, w_ref, b_ref, o_ref):
    o_ref[...] = jnp.dot(x_ref[...], w_ref[...],
                         preferred_element_type=jnp.float32) + b_ref[...]


def _root(x, w_root, b):
    return pl.pallas_call(
        _root_body,
        grid=(N_NODES // _BN,),
        in_specs=[
            pl.BlockSpec((_BN, D), lambda i: (i, 0)),
            pl.BlockSpec((D, D), lambda i: (0, 0)),
            pl.BlockSpec((1, D), lambda i: (0, 0)),
        ],
        out_specs=pl.BlockSpec((_BN, D), lambda i: (i, 0)),
        out_shape=jax.ShapeDtypeStruct((N_NODES, D), jnp.float32),
    )(x, w_root, b.reshape(1, D))


def _post_body(p_ref, r_ref, wrel_ref, o_ref, *, relu):
    agg = p_ref[0] + p_ref[1]
    acc = jnp.dot(agg, wrel_ref[...],
                  preferred_element_type=jnp.float32) + r_ref[...]
    o_ref[...] = jnp.maximum(acc, 0.0) if relu else acc


def _post(partials, root, w_rel, relu):
    return pl.pallas_call(
        functools.partial(_post_body, relu=relu),
        grid=(N_NODES // _BN,),
        in_specs=[
            pl.BlockSpec((NC, _BN, D), lambda i: (0, i, 0)),
            pl.BlockSpec((_BN, D), lambda i: (i, 0)),
            pl.BlockSpec((D, D), lambda i: (0, 0)),
        ],
        out_specs=pl.BlockSpec((_BN, D), lambda i: (i, 0)),
        out_shape=jax.ShapeDtypeStruct((N_NODES, D), jnp.float32),
    )(partials, root, w_rel)


def kernel(x, edge_index, W1_rel, b1, W1_root, W2_rel, b2, W2_root):
    ei = edge_index.astype(jnp.int32)
    src = ei[0].reshape(NW, SCH, CSC, CH)
    dst = ei[1].reshape(NW, SCH, CSC, CH)
    p1 = _sc_agg(x, src, dst)
    root1 = _root(x, W1_root, b1)
    h = _post(p1, root1, W1_rel, relu=True)
    p2 = _sc_agg(h, src, dst)
    root2 = _root(h, W2_root, b2)
    out = _post(p2, root2, W2_rel, relu=False)
    return out


# speedup vs baseline: 1.3300x; 1.0355x over previous
import functools

import jax
import jax.numpy as jnp
from jax import lax
from jax.experimental import pallas as pl
from jax.experimental.pallas import tpu as pltpu
from jax.experimental.pallas import tpu_sc as plsc

N_NODES = 10000
N_EDGES = 320000
D = 128

NC = 2
NS = 16
NW = NC * NS
EPW = N_EDGES // NW
CH = 40
NCH = EPW // CH
NB = 6
CSC = 25
SCH = NCH // CSC
NITER = -(-CSC // NB)
RPT = 624
TAIL = N_NODES - RPT * NS
ZR = 16

_mesh = plsc.VectorSubcoreMesh(core_axis_name="c", subcore_axis_name="s")


@functools.partial(
    pl.kernel,
    out_type=jax.ShapeDtypeStruct((NC, N_NODES, D), jnp.float32),
    mesh=_mesh,
    scratch_types=[
        pltpu.VMEM((2 * CSC * CH,), jnp.int32),
        pltpu.VMEM((2 * CSC * CH,), jnp.int32),
        pltpu.VMEM((NB, CH, D), jnp.float32),
        pltpu.VMEM((ZR, D), jnp.float32),
        pltpu.VMEM_SHARED((N_NODES, D), jnp.float32),
        pltpu.SemaphoreType.DMA((NB,)),
        pltpu.SemaphoreType.DMA((NB,)),
        pltpu.SemaphoreType.DMA,
        pltpu.SemaphoreType.DMA,
    ],
)
def _sc_agg(table, src, dst, out, sbuf, dbuf, rows, zbuf, acc, gsem, ssem,
            zsem, isem):
    cid = lax.axis_index("c")
    sid = lax.axis_index("s")
    wid = sid * NC + cid

    zv = jnp.zeros((16,), jnp.float32)
    for i in range(ZR):
        for j in range(D // 16):
            zbuf[i, pl.ds(j * 16, 16)] = zv

    def zero_issue(i, carry):
        pltpu.async_copy(zbuf, acc.at[pl.ds(sid * RPT + i * ZR, ZR)], zsem)
        return carry

    lax.fori_loop(0, RPT // ZR, zero_issue, 0)

    @pl.when(sid == 0)
    def _zero_tail():
        pltpu.async_copy(zbuf.at[pl.ds(0, TAIL)],
                         acc.at[pl.ds(RPT * NS, TAIL)], zsem)

    pltpu.sync_copy(src.at[pl.ds(wid * EPW, CSC * CH)],
                    sbuf.at[pl.ds(0, CSC * CH)])
    pltpu.sync_copy(dst.at[pl.ds(wid * EPW, CSC * CH)],
                    dbuf.at[pl.ds(0, CSC * CH)])
    for b in range(NB):
        pltpu.async_copy(table.at[sbuf.at[pl.ds(b * CH, CH)]], rows.at[b],
                         gsem.at[b])

    def zero_drain(i, carry):
        pltpu.make_async_copy(zbuf, acc.at[pl.ds(sid * RPT + i * ZR, ZR)],
                              zsem).wait()
        return carry

    lax.fori_loop(0, RPT // ZR, zero_drain, 0)

    @pl.when(sid == 0)
    def _zero_tail_drain():
        pltpu.make_async_copy(zbuf.at[pl.ds(0, TAIL)],
                              acc.at[pl.ds(RPT * NS, TAIL)], zsem).wait()

    plsc.subcore_barrier()

    def super_body(s, carry):
        p = (s % 2) * CSC * CH
        q = ((s + 1) % 2) * CSC * CH
        nbase = wid * EPW + (s + 1) * CSC * CH

        @pl.when(s + 1 < SCH)
        def _stage_next():
            pltpu.async_copy(src.at[pl.ds(nbase, CSC * CH)],
                             sbuf.at[pl.ds(q, CSC * CH)], isem)
            pltpu.async_copy(dst.at[pl.ds(nbase, CSC * CH)],
                             dbuf.at[pl.ds(q, CSC * CH)], isem)

        def main(i, carry2):
            for b in range(NB):
                j = i * NB + b

                @pl.when(j < CSC)
                def _scatter(b=b, j=j):
                    pltpu.make_async_copy(
                        table.at[sbuf.at[pl.ds(p + j * CH, CH)]], rows.at[b],
                        gsem.at[b]).wait()
                    pltpu.async_copy(rows.at[b],
                                     acc.at[dbuf.at[pl.ds(p + j * CH, CH)]],
                                     ssem.at[b], add=True)

            for b in range(NB):
                j = i * NB + b

                @pl.when(j < CSC)
                def _drain(b=b, j=j):
                    pltpu.make_async_copy(
                        rows.at[b], acc.at[dbuf.at[pl.ds(p + j * CH, CH)]],
                        ssem.at[b]).wait()

                @pl.when(j + NB < CSC)
                def _prefetch(b=b, j=j):
                    pltpu.async_copy(
                        table.at[sbuf.at[pl.ds(p + (j + NB) * CH, CH)]],
                        rows.at[b], gsem.at[b])

            return carry2

        lax.fori_loop(0, NITER, main, 0)

        @pl.when(s + 1 < SCH)
        def _prime_next():
            pltpu.make_async_copy(src.at[pl.ds(nbase, CSC * CH)],
                                  sbuf.at[pl.ds(q, CSC * CH)], isem).wait()
            pltpu.make_async_copy(dst.at[pl.ds(nbase, CSC * CH)],
                                  dbuf.at[pl.ds(q, CSC * CH)], isem).wait()
            for b in range(NB):
                pltpu.async_copy(table.at[sbuf.at[pl.ds(q + b * CH, CH)]],
                                 rows.at[b], gsem.at[b])
        return carry

    lax.fori_loop(0, SCH, super_body, 0)
    plsc.subcore_barrier()

    pltpu.sync_copy(acc.at[pl.ds(sid * RPT, RPT)],
                    out.at[cid, pl.ds(sid * RPT, RPT)])

    @pl.when(sid == 0)
    def _copy_tail():
        pltpu.sync_copy(acc.at[pl.ds(RPT * NS, TAIL)],
                        out.at[cid, pl.ds(RPT * NS, TAIL)])


_BN = 2000


def _root_body(x_ref, w_ref, b_ref, o_ref):
    o_ref[...] = jnp.dot(x_ref[...], w_ref[...],
                         preferred_element_type=jnp.float32) + b_ref[...]


def _root(x, w_root, b):
    return pl.pallas_call(
        _root_body,
        grid=(N_NODES // _BN,),
        in_specs=[
            pl.BlockSpec((_BN, D), lambda i: (i, 0)),
            pl.BlockSpec((D, D), lambda i: (0, 0)),
            pl.BlockSpec((1, D), lambda i: (0, 0)),
        ],
        out_specs=pl.BlockSpec((_BN, D), lambda i: (i, 0)),
        out_shape=jax.ShapeDtypeStruct((N_NODES, D), jnp.float32),
    )(x, w_root, b.reshape(1, D))


def _post_body(p_ref, r_ref, wrel_ref, o_ref, *, relu):
    agg = p_ref[0] + p_ref[1]
    acc = jnp.dot(agg, wrel_ref[...],
                  preferred_element_type=jnp.float32) + r_ref[...]
    o_ref[...] = jnp.maximum(acc, 0.0) if relu else acc


def _post(partials, root, w_rel, relu):
    return pl.pallas_call(
        functools.partial(_post_body, relu=relu),
        grid=(N_NODES // _BN,),
        in_specs=[
            pl.BlockSpec((NC, _BN, D), lambda i: (0, i, 0)),
            pl.BlockSpec((_BN, D), lambda i: (i, 0)),
            pl.BlockSpec((D, D), lambda i: (0, 0)),
        ],
        out_specs=pl.BlockSpec((_BN, D), lambda i: (i, 0)),
        out_shape=jax.ShapeDtypeStruct((N_NODES, D), jnp.float32),
    )(partials, root, w_rel)


def kernel(x, edge_index, W1_rel, b1, W1_root, W2_rel, b2, W2_root):
    ei = edge_index.astype(jnp.int32)
    src = ei[0]
    dst = ei[1]
    p1 = _sc_agg(x, src, dst)
    root1 = _root(x, W1_root, b1)
    h = _post(p1, root1, W1_rel, relu=True)
    p2 = _sc_agg(h, src, dst)
    root2 = _root(h, W2_root, b2)
    out = _post(p2, root2, W2_rel, relu=False)
    return out
